# baseline (speedup 1.0000x reference)
import functools

import jax
import jax.numpy as jnp
from jax import lax
from jax.experimental import pallas as pl
from jax.experimental.pallas import tpu as pltpu
from jax.experimental.pallas import tpu_sc as plsc

N = 10000
E = 320000
DF = 128
DH = 16
NP = 10240
NPP = 10112

NC = 2
NS = 16
NW = NC * NS
EPT = E // NW
B = 1000
NBF = EPT // B
TAIL = EPT - NBF * B
RPS = NP // NS
RPW = NP // NW
PIPE = 4

_mesh = plsc.VectorSubcoreMesh(core_axis_name="c", subcore_axis_name="s")
_sc_params = pltpu.CompilerParams(needs_layout_passes=False,
                                  use_tc_tiling_on_sc=False)


@functools.partial(
    pl.kernel,
    out_type=jax.ShapeDtypeStruct((NW * NPP,), jnp.float32),
    mesh=_mesh,
    compiler_params=_sc_params,
    scratch_types=[
        pltpu.VMEM((EPT,), jnp.int32),
        pltpu.VMEM((NPP,), jnp.float32),
    ],
)
def _sc_degree(edges_hbm, out_hbm, idx_v, deg_v):
    c = lax.axis_index("c")
    s = lax.axis_index("s")
    wid = c * NS + s
    pltpu.sync_copy(edges_hbm.at[1, pl.ds(wid * EPT, EPT)], idx_v)
    zeros = jnp.zeros((16,), jnp.float32)
    ones = jnp.ones((16,), jnp.float32)

    def zero_body(i, carry):
        deg_v[pl.ds(i * 16, 16)] = zeros
        return carry

    lax.fori_loop(0, NPP // 16, zero_body, 0, unroll=4)

    def edge_body(i, carry):
        dv = idx_v[pl.ds(i * 16, 16)]
        plsc.addupdate_scatter(deg_v, [dv], ones)
        return carry

    lax.fori_loop(0, EPT // 16, edge_body, 0, unroll=4)
    pltpu.sync_copy(deg_v, out_hbm.at[pl.ds(wid * NPP, NPP)])


@functools.partial(
    pl.kernel,
    out_type=jax.ShapeDtypeStruct((NC, NP, DH), jnp.float32),
    mesh=_mesh,
    compiler_params=_sc_params,
    scratch_types=[
        pltpu.VMEM((EPT,), jnp.int32),
        pltpu.VMEM((EPT,), jnp.int32),
        pltpu.VMEM((PIPE, B, DH), jnp.float32),
        pltpu.VMEM((max(TAIL, 16), DH), jnp.float32),
        pltpu.VMEM_SHARED((NP, DH), jnp.float32),
        pltpu.SemaphoreType.DMA((PIPE,)),
        pltpu.SemaphoreType.DMA((PIPE,)),
    ],
)
def _sc_scatter(y_hbm, edges_hbm, zero_hbm, out_hbm,
                src_v, dst_v, rows_v, tail_v, acc_sh, gsem, wsem):
    c = lax.axis_index("c")
    s = lax.axis_index("s")
    wid = c * NS + s

    pltpu.sync_copy(edges_hbm.at[0, pl.ds(wid * EPT, EPT)], src_v)
    pltpu.sync_copy(edges_hbm.at[1, pl.ds(wid * EPT, EPT)], dst_v)
    pltpu.sync_copy(zero_hbm.at[pl.ds(s * RPS, RPS)],
                    acc_sh.at[pl.ds(s * RPS, RPS)])
    plsc.subcore_barrier()

    def start_gather(j):
        slot = j % PIPE
        return pltpu.async_copy(y_hbm.at[src_v.at[pl.ds(j * B, B)]],
                                rows_v.at[slot], gsem.at[slot])

    def start_scatter(j):
        slot = j % PIPE
        return pltpu.async_copy(rows_v.at[slot],
                                acc_sh.at[dst_v.at[pl.ds(j * B, B)]],
                                wsem.at[slot], add=True)

    gathers = {}
    scatters = {}
    for j in range(min(PIPE, NBF)):
        gathers[j] = start_gather(j)
    for j in range(NBF):
        gathers.pop(j).wait()
        scatters[j] = start_scatter(j)
        nj = j + PIPE
        if nj < NBF:
            scatters.pop(nj - PIPE).wait()
            gathers[nj] = start_gather(nj)
    if TAIL:
        pltpu.async_copy(y_hbm.at[src_v.at[pl.ds(NBF * B, TAIL)]], tail_v,
                         gsem.at[0]).wait()
        pltpu.sync_copy(tail_v, acc_sh.at[dst_v.at[pl.ds(NBF * B, TAIL)]],
                        add=True)
    for j in list(scatters):
        scatters.pop(j).wait()

    plsc.subcore_barrier()
    pltpu.sync_copy(acc_sh.at[pl.ds(s * RPS, RPS)],
                    out_hbm.at[c, pl.ds(s * RPS, RPS)])


@functools.partial(
    pl.kernel,
    out_type=[jax.ShapeDtypeStruct((NP, DH), jnp.float32),
              jax.ShapeDtypeStruct((NP, DH), jnp.float32)],
    mesh=_mesh,
    compiler_params=_sc_params,
    scratch_types=[
        pltpu.VMEM((RPW, DH), jnp.float32),
        pltpu.VMEM((RPW, DH), jnp.float32),
        pltpu.VMEM((RPW, DH), jnp.float32),
        pltpu.VMEM((RPW,), jnp.float32),
        pltpu.VMEM((DH,), jnp.float32),
        pltpu.VMEM((RPW, DH), jnp.float32),
        pltpu.VMEM((RPW, DH), jnp.float32),
    ],
)
def _sc_mid(p_hbm, y1_hbm, d_hbm, b1_hbm, out_hbm, dexp_hbm,
            p0_v, p1_v, y1_v, d_v, b1_v, y2_v, dexp_v):
    c = lax.axis_index("c")
    s = lax.axis_index("s")
    wid = c * NS + s
    base = wid * RPW
    pltpu.sync_copy(p_hbm.at[0, pl.ds(base, RPW)], p0_v)
    pltpu.sync_copy(p_hbm.at[1, pl.ds(base, RPW)], p1_v)
    pltpu.sync_copy(y1_hbm.at[pl.ds(base, RPW)], y1_v)
    pltpu.sync_copy(d_hbm.at[pl.ds(base, RPW)], d_v)
    pltpu.sync_copy(b1_hbm, b1_v)
    b1 = b1_v[pl.ds(0, DH)]

    def row_body(r, carry):
        agg = p0_v[r] + p1_v[r] + y1_v[r]
        dval = plsc.load_gather(d_v, [jnp.full((16,), r, jnp.int32)])
        h = jnp.maximum(agg * dval + b1, 0.0)
        y2_v[r] = h * dval
        dexp_v[r] = dval
        return carry

    lax.fori_loop(0, RPW, row_body, 0, unroll=4)
    pltpu.sync_copy(y2_v, out_hbm.at[pl.ds(base, RPW)])
    pltpu.sync_copy(dexp_v, dexp_hbm.at[pl.ds(base, RPW)])


def _tc_matmul(x, W1):
    def body(x_ref, w_ref, out_ref):
        out_ref[...] = jnp.dot(x_ref[...], w_ref[...],
                               preferred_element_type=jnp.float32)

    return pl.pallas_call(
        body,
        out_shape=jax.ShapeDtypeStruct((N, DH), jnp.float32),
    )(x, W1)


def _tc_scale(degp, xw):
    def body(degp_ref, xw_ref, y_ref, d_ref):
        deg = jnp.full((N,), 1.0, jnp.float32)
        for w in range(NW):
            deg = deg + degp_ref[pl.ds(w * NPP, N)]
        d = lax.rsqrt(deg)
        y = xw_ref[...] * d[:, None]
        d_ref[...] = jnp.concatenate([d, jnp.ones((NP - N,), jnp.float32)])
        y_ref[...] = jnp.concatenate(
            [y, jnp.zeros((NP - N, DH), jnp.float32)], axis=0)

    return pl.pallas_call(
        body,
        out_shape=[jax.ShapeDtypeStruct((NP, DH), jnp.float32),
                   jax.ShapeDtypeStruct((NP,), jnp.float32)],
    )(degp, xw)


def _tc_out(q128, y2_128, dexp128, W2exp, b2exp):
    NR8 = N // 8

    def body(q_ref, y2_ref, de_ref, w_ref, b_ref, out_ref):
        t = (q_ref[0, :NR8] + q_ref[1, :NR8] + y2_ref[:NR8]) * de_ref[:NR8]
        for k in range(8):
            out_ref[:, k, :] = (
                jnp.dot(t, w_ref[:, k * DF:(k + 1) * DF],
                        preferred_element_type=jnp.float32)
                + b_ref[pl.ds(k * DF, DF)][None, :])

    return pl.pallas_call(
        body,
        out_shape=jax.ShapeDtypeStruct((NR8, 8, DF), jnp.float32),
    )(q128, y2_128, dexp128, W2exp, b2exp)


def kernel(x, edge_index, W1, b1, W2, b2):
    edges = edge_index.astype(jnp.int32)
    zeros = jnp.zeros((NP, DH), jnp.float32)

    xw = _tc_matmul(x, W1)
    degp = _sc_degree(edges)
    y1, d = _tc_scale(degp, xw)
    p = _sc_scatter(y1, edges, zeros)
    y2, dexp = _sc_mid(p, y1, d, b1)
    q = _sc_scatter(y2, edges, zeros)

    q128 = q.reshape(NC, NP // 8, 8 * DH)
    y2_128 = y2.reshape(NP // 8, 8 * DH)
    dexp128 = dexp.reshape(NP // 8, 8 * DH)
    W2exp = (jnp.eye(8, dtype=jnp.float32)[:, None, :, None]
             * W2[None, :, None, :]).reshape(8 * DH, 8 * DF)
    b2exp = jnp.tile(b2, 8)

    out = _tc_out(q128, y2_128, dexp128, W2exp, b2exp)
    return out.reshape(N, DF)


# speedup vs baseline: 1.0011x; 1.0011x over previous
import functools

import jax
import jax.numpy as jnp
from jax import lax
from jax.experimental import pallas as pl
from jax.experimental.pallas import tpu as pltpu
from jax.experimental.pallas import tpu_sc as plsc

N = 10000
E = 320000
DF = 128
DH = 16
NP = 10240
NPP = 10112

NC = 2
NS = 16
NW = NC * NS
EPT = E // NW
B = 400
NBF = EPT // B
TAIL = EPT - NBF * B
RPS = NP // NS
RPW = NP // NW
PIPE = 8

_mesh = plsc.VectorSubcoreMesh(core_axis_name="c", subcore_axis_name="s")
_sc_params = pltpu.CompilerParams(needs_layout_passes=False,
                                  use_tc_tiling_on_sc=False)


@functools.partial(
    pl.kernel,
    out_type=jax.ShapeDtypeStruct((NW * NPP,), jnp.float32),
    mesh=_mesh,
    compiler_params=_sc_params,
    scratch_types=[
        pltpu.VMEM((EPT,), jnp.int32),
        pltpu.VMEM((NPP,), jnp.float32),
    ],
)
def _sc_degree(edges_hbm, out_hbm, idx_v, deg_v):
    c = lax.axis_index("c")
    s = lax.axis_index("s")
    wid = c * NS + s
    pltpu.sync_copy(edges_hbm.at[1, pl.ds(wid * EPT, EPT)], idx_v)
    zeros = jnp.zeros((16,), jnp.float32)
    ones = jnp.ones((16,), jnp.float32)

    def zero_body(i, carry):
        deg_v[pl.ds(i * 16, 16)] = zeros
        return carry

    lax.fori_loop(0, NPP // 16, zero_body, 0, unroll=4)

    def edge_body(i, carry):
        dv = idx_v[pl.ds(i * 16, 16)]
        plsc.addupdate_scatter(deg_v, [dv], ones)
        return carry

    lax.fori_loop(0, EPT // 16, edge_body, 0, unroll=4)
    pltpu.sync_copy(deg_v, out_hbm.at[pl.ds(wid * NPP, NPP)])


@functools.partial(
    pl.kernel,
    out_type=jax.ShapeDtypeStruct((NC, NP, DH), jnp.float32),
    mesh=_mesh,
    compiler_params=_sc_params,
    scratch_types=[
        pltpu.VMEM((EPT,), jnp.int32),
        pltpu.VMEM((EPT,), jnp.int32),
        pltpu.VMEM((PIPE, B, DH), jnp.float32),
        pltpu.VMEM((max(TAIL, 16), DH), jnp.float32),
        pltpu.VMEM_SHARED((NP, DH), jnp.float32),
        pltpu.SemaphoreType.DMA((PIPE,)),
        pltpu.SemaphoreType.DMA((PIPE,)),
    ],
)
def _sc_scatter(y_hbm, edges_hbm, zero_hbm, out_hbm,
                src_v, dst_v, rows_v, tail_v, acc_sh, gsem, wsem):
    c = lax.axis_index("c")
    s = lax.axis_index("s")
    wid = c * NS + s

    pltpu.sync_copy(edges_hbm.at[0, pl.ds(wid * EPT, EPT)], src_v)
    pltpu.sync_copy(edges_hbm.at[1, pl.ds(wid * EPT, EPT)], dst_v)
    pltpu.sync_copy(zero_hbm.at[pl.ds(s * RPS, RPS)],
                    acc_sh.at[pl.ds(s * RPS, RPS)])
    plsc.subcore_barrier()

    def start_gather(j):
        slot = j % PIPE
        return pltpu.async_copy(y_hbm.at[src_v.at[pl.ds(j * B, B)]],
                                rows_v.at[slot], gsem.at[slot])

    def start_scatter(j):
        slot = j % PIPE
        return pltpu.async_copy(rows_v.at[slot],
                                acc_sh.at[dst_v.at[pl.ds(j * B, B)]],
                                wsem.at[slot], add=True)

    gathers = {}
    scatters = {}
    for j in range(min(PIPE, NBF)):
        gathers[j] = start_gather(j)
    for j in range(NBF):
        gathers.pop(j).wait()
        scatters[j] = start_scatter(j)
        nj = j + PIPE
        if nj < NBF:
            scatters.pop(nj - PIPE).wait()
            gathers[nj] = start_gather(nj)
    if TAIL:
        pltpu.async_copy(y_hbm.at[src_v.at[pl.ds(NBF * B, TAIL)]], tail_v,
                         gsem.at[0]).wait()
        pltpu.sync_copy(tail_v, acc_sh.at[dst_v.at[pl.ds(NBF * B, TAIL)]],
                        add=True)
    for j in list(scatters):
        scatters.pop(j).wait()

    plsc.subcore_barrier()
    pltpu.sync_copy(acc_sh.at[pl.ds(s * RPS, RPS)],
                    out_hbm.at[c, pl.ds(s * RPS, RPS)])


@functools.partial(
    pl.kernel,
    out_type=[jax.ShapeDtypeStruct((NP, DH), jnp.float32),
              jax.ShapeDtypeStruct((NP, DH), jnp.float32)],
    mesh=_mesh,
    compiler_params=_sc_params,
    scratch_types=[
        pltpu.VMEM((RPW, DH), jnp.float32),
        pltpu.VMEM((RPW, DH), jnp.float32),
        pltpu.VMEM((RPW, DH), jnp.float32),
        pltpu.VMEM((RPW,), jnp.float32),
        pltpu.VMEM((DH,), jnp.float32),
        pltpu.VMEM((RPW, DH), jnp.float32),
        pltpu.VMEM((RPW, DH), jnp.float32),
    ],
)
def _sc_mid(p_hbm, y1_hbm, d_hbm, b1_hbm, out_hbm, dexp_hbm,
            p0_v, p1_v, y1_v, d_v, b1_v, y2_v, dexp_v):
    c = lax.axis_index("c")
    s = lax.axis_index("s")
    wid = c * NS + s
    base = wid * RPW
    pltpu.sync_copy(p_hbm.at[0, pl.ds(base, RPW)], p0_v)
    pltpu.sync_copy(p_hbm.at[1, pl.ds(base, RPW)], p1_v)
    pltpu.sync_copy(y1_hbm.at[pl.ds(base, RPW)], y1_v)
    pltpu.sync_copy(d_hbm.at[pl.ds(base, RPW)], d_v)
    pltpu.sync_copy(b1_hbm, b1_v)
    b1 = b1_v[pl.ds(0, DH)]

    def row_body(r, carry):
        agg = p0_v[r] + p1_v[r] + y1_v[r]
        dval = plsc.load_gather(d_v, [jnp.full((16,), r, jnp.int32)])
        h = jnp.maximum(agg * dval + b1, 0.0)
        y2_v[r] = h * dval
        dexp_v[r] = dval
        return carry

    lax.fori_loop(0, RPW, row_body, 0, unroll=4)
    pltpu.sync_copy(y2_v, out_hbm.at[pl.ds(base, RPW)])
    pltpu.sync_copy(dexp_v, dexp_hbm.at[pl.ds(base, RPW)])


def _tc_matmul(x, W1):
    def body(x_ref, w_ref, out_ref):
        out_ref[...] = jnp.dot(x_ref[...], w_ref[...],
                               preferred_element_type=jnp.float32)

    return pl.pallas_call(
        body,
        out_shape=jax.ShapeDtypeStruct((N, DH), jnp.float32),
    )(x, W1)


def _tc_scale(degp, xw):
    def body(degp_ref, xw_ref, y_ref, d_ref):
        deg = jnp.full((N,), 1.0, jnp.float32)
        for w in range(NW):
            deg = deg + degp_ref[pl.ds(w * NPP, N)]
        d = lax.rsqrt(deg)
        y = xw_ref[...] * d[:, None]
        d_ref[...] = jnp.concatenate([d, jnp.ones((NP - N,), jnp.float32)])
        y_ref[...] = jnp.concatenate(
            [y, jnp.zeros((NP - N, DH), jnp.float32)], axis=0)

    return pl.pallas_call(
        body,
        out_shape=[jax.ShapeDtypeStruct((NP, DH), jnp.float32),
                   jax.ShapeDtypeStruct((NP,), jnp.float32)],
    )(degp, xw)


def _tc_out(q128, y2_128, dexp128, W2exp, b2exp):
    NR8 = N // 8

    def body(q_ref, y2_ref, de_ref, w_ref, b_ref, out_ref):
        t = (q_ref[0, :NR8] + q_ref[1, :NR8] + y2_ref[:NR8]) * de_ref[:NR8]
        for k in range(8):
            out_ref[:, k, :] = (
                jnp.dot(t, w_ref[:, k * DF:(k + 1) * DF],
                        preferred_element_type=jnp.float32)
                + b_ref[pl.ds(k * DF, DF)][None, :])

    return pl.pallas_call(
        body,
        out_shape=jax.ShapeDtypeStruct((NR8, 8, DF), jnp.float32),
    )(q128, y2_128, dexp128, W2exp, b2exp)


def kernel(x, edge_index, W1, b1, W2, b2):
    edges = edge_index.astype(jnp.int32)
    zeros = jnp.zeros((NP, DH), jnp.float32)

    xw = _tc_matmul(x, W1)
    degp = _sc_degree(edges)
    y1, d = _tc_scale(degp, xw)
    p = _sc_scatter(y1, edges, zeros)
    y2, dexp = _sc_mid(p, y1, d, b1)
    q = _sc_scatter(y2, edges, zeros)

    q128 = q.reshape(NC, NP // 8, 8 * DH)
    y2_128 = y2.reshape(NP // 8, 8 * DH)
    dexp128 = dexp.reshape(NP // 8, 8 * DH)
    W2exp = (jnp.eye(8, dtype=jnp.float32)[:, None, :, None]
             * W2[None, :, None, :]).reshape(8 * DH, 8 * DF)
    b2exp = jnp.tile(b2, 8)

    out = _tc_out(q128, y2_128, dexp128, W2exp, b2exp)
    return out.reshape(N, DF)


# speedup vs baseline: 1.0131x; 1.0120x over previous
import functools

import jax
import jax.numpy as jnp
from jax import lax
from jax.experimental import pallas as pl
from jax.experimental.pallas import tpu as pltpu
from jax.experimental.pallas import tpu_sc as plsc

N = 10000
E = 320000
DF = 128
DH = 16
NP = 10240
NPP = 10112

NC = 2
NS = 16
NW = NC * NS
EPT = E // NW
B = 1000
NBF = EPT // B
TAIL = EPT - NBF * B
RPS = NP // NS
RPW = NP // NW
PIPE = 5

_mesh = plsc.VectorSubcoreMesh(core_axis_name="c", subcore_axis_name="s")
_sc_params = pltpu.CompilerParams(needs_layout_passes=False,
                                  use_tc_tiling_on_sc=False)


@functools.partial(
    pl.kernel,
    out_type=jax.ShapeDtypeStruct((NW * NPP,), jnp.float32),
    mesh=_mesh,
    compiler_params=_sc_params,
    scratch_types=[
        pltpu.VMEM((EPT,), jnp.int32),
        pltpu.VMEM((NPP,), jnp.float32),
    ],
)
def _sc_degree(edges_hbm, out_hbm, idx_v, deg_v):
    c = lax.axis_index("c")
    s = lax.axis_index("s")
    wid = c * NS + s
    pltpu.sync_copy(edges_hbm.at[1, pl.ds(wid * EPT, EPT)], idx_v)
    zeros = jnp.zeros((16,), jnp.float32)
    ones = jnp.ones((16,), jnp.float32)

    def zero_body(i, carry):
        deg_v[pl.ds(i * 16, 16)] = zeros
        return carry

    lax.fori_loop(0, NPP // 16, zero_body, 0, unroll=4)

    def edge_body(i, carry):
        dv = idx_v[pl.ds(i * 16, 16)]
        plsc.addupdate_scatter(deg_v, [dv], ones)
        return carry

    lax.fori_loop(0, EPT // 16, edge_body, 0, unroll=4)
    pltpu.sync_copy(deg_v, out_hbm.at[pl.ds(wid * NPP, NPP)])


@functools.partial(
    pl.kernel,
    out_type=jax.ShapeDtypeStruct((NC, NP, DH), jnp.float32),
    mesh=_mesh,
    compiler_params=_sc_params,
    scratch_types=[
        pltpu.VMEM((EPT,), jnp.int32),
        pltpu.VMEM((EPT,), jnp.int32),
        pltpu.VMEM((PIPE, B, DH), jnp.float32),
        pltpu.VMEM((max(TAIL, 16), DH), jnp.float32),
        pltpu.VMEM_SHARED((NP, DH), jnp.float32),
        pltpu.SemaphoreType.DMA((PIPE,)),
        pltpu.SemaphoreType.DMA((PIPE,)),
    ],
)
def _sc_scatter(y_hbm, edges_hbm, zero_hbm, out_hbm,
                src_v, dst_v, rows_v, tail_v, acc_sh, gsem, wsem):
    c = lax.axis_index("c")
    s = lax.axis_index("s")
    wid = c * NS + s

    pltpu.sync_copy(edges_hbm.at[0, pl.ds(wid * EPT, EPT)], src_v)
    pltpu.sync_copy(edges_hbm.at[1, pl.ds(wid * EPT, EPT)], dst_v)
    pltpu.sync_copy(zero_hbm.at[pl.ds(s * RPS, RPS)],
                    acc_sh.at[pl.ds(s * RPS, RPS)])
    plsc.subcore_barrier()

    def start_gather(j):
        slot = j % PIPE
        return pltpu.async_copy(y_hbm.at[src_v.at[pl.ds(j * B, B)]],
                                rows_v.at[slot], gsem.at[slot])

    def start_scatter(j):
        slot = j % PIPE
        return pltpu.async_copy(rows_v.at[slot],
                                acc_sh.at[dst_v.at[pl.ds(j * B, B)]],
                                wsem.at[slot], add=True)

    gathers = {}
    scatters = {}
    for j in range(min(PIPE, NBF)):
        gathers[j] = start_gather(j)
    for j in range(NBF):
        gathers.pop(j).wait()
        scatters[j] = start_scatter(j)
        nj = j + PIPE
        if nj < NBF:
            scatters.pop(nj - PIPE).wait()
            gathers[nj] = start_gather(nj)
    if TAIL:
        pltpu.async_copy(y_hbm.at[src_v.at[pl.ds(NBF * B, TAIL)]], tail_v,
                         gsem.at[0]).wait()
        pltpu.sync_copy(tail_v, acc_sh.at[dst_v.at[pl.ds(NBF * B, TAIL)]],
                        add=True)
    for j in list(scatters):
        scatters.pop(j).wait()

    plsc.subcore_barrier()
    pltpu.sync_copy(acc_sh.at[pl.ds(s * RPS, RPS)],
                    out_hbm.at[c, pl.ds(s * RPS, RPS)])


@functools.partial(
    pl.kernel,
    out_type=[jax.ShapeDtypeStruct((NP, DH), jnp.float32),
              jax.ShapeDtypeStruct((NP, DH), jnp.float32)],
    mesh=_mesh,
    compiler_params=_sc_params,
    scratch_types=[
        pltpu.VMEM((RPW, DH), jnp.float32),
        pltpu.VMEM((RPW, DH), jnp.float32),
        pltpu.VMEM((RPW, DH), jnp.float32),
        pltpu.VMEM((RPW,), jnp.float32),
        pltpu.VMEM((DH,), jnp.float32),
        pltpu.VMEM((RPW, DH), jnp.float32),
        pltpu.VMEM((RPW, DH), jnp.float32),
    ],
)
def _sc_mid(p_hbm, y1_hbm, d_hbm, b1_hbm, out_hbm, dexp_hbm,
            p0_v, p1_v, y1_v, d_v, b1_v, y2_v, dexp_v):
    c = lax.axis_index("c")
    s = lax.axis_index("s")
    wid = c * NS + s
    base = wid * RPW
    pltpu.sync_copy(p_hbm.at[0, pl.ds(base, RPW)], p0_v)
    pltpu.sync_copy(p_hbm.at[1, pl.ds(base, RPW)], p1_v)
    pltpu.sync_copy(y1_hbm.at[pl.ds(base, RPW)], y1_v)
    pltpu.sync_copy(d_hbm.at[pl.ds(base, RPW)], d_v)
    pltpu.sync_copy(b1_hbm, b1_v)
    b1 = b1_v[pl.ds(0, DH)]

    def row_body(r, carry):
        agg = p0_v[r] + p1_v[r] + y1_v[r]
        dval = plsc.load_gather(d_v, [jnp.full((16,), r, jnp.int32)])
        h = jnp.maximum(agg * dval + b1, 0.0)
        y2_v[r] = h * dval
        dexp_v[r] = dval
        return carry

    lax.fori_loop(0, RPW, row_body, 0, unroll=4)
    pltpu.sync_copy(y2_v, out_hbm.at[pl.ds(base, RPW)])
    pltpu.sync_copy(dexp_v, dexp_hbm.at[pl.ds(base, RPW)])


def _tc_matmul(x, W1):
    def body(x_ref, w_ref, out_ref):
        out_ref[...] = jnp.dot(x_ref[...], w_ref[...],
                               preferred_element_type=jnp.float32)

    return pl.pallas_call(
        body,
        out_shape=jax.ShapeDtypeStruct((N, DH), jnp.float32),
    )(x, W1)


def _tc_scale(degp, xw):
    def body(degp_ref, xw_ref, y_ref, d_ref):
        deg = jnp.full((N,), 1.0, jnp.float32)
        for w in range(NW):
            deg = deg + degp_ref[pl.ds(w * NPP, N)]
        d = lax.rsqrt(deg)
        y = xw_ref[...] * d[:, None]
        d_ref[...] = jnp.concatenate([d, jnp.ones((NP - N,), jnp.float32)])
        y_ref[...] = jnp.concatenate(
            [y, jnp.zeros((NP - N, DH), jnp.float32)], axis=0)

    return pl.pallas_call(
        body,
        out_shape=[jax.ShapeDtypeStruct((NP, DH), jnp.float32),
                   jax.ShapeDtypeStruct((NP,), jnp.float32)],
    )(degp, xw)


def _tc_out(q128, y2_128, dexp128, W2exp, b2exp):
    NR8 = N // 8

    def body(q_ref, y2_ref, de_ref, w_ref, b_ref, out_ref):
        t = (q_ref[0, :NR8] + q_ref[1, :NR8] + y2_ref[:NR8]) * de_ref[:NR8]
        for k in range(8):
            out_ref[:, k, :] = (
                jnp.dot(t, w_ref[:, k * DF:(k + 1) * DF],
                        preferred_element_type=jnp.float32)
                + b_ref[pl.ds(k * DF, DF)][None, :])

    return pl.pallas_call(
        body,
        out_shape=jax.ShapeDtypeStruct((NR8, 8, DF), jnp.float32),
    )(q128, y2_128, dexp128, W2exp, b2exp)


def kernel(x, edge_index, W1, b1, W2, b2):
    edges = edge_index.astype(jnp.int32)
    zeros = jnp.zeros((NP, DH), jnp.float32)

    xw = _tc_matmul(x, W1)
    degp = _sc_degree(edges)
    y1, d = _tc_scale(degp, xw)
    p = _sc_scatter(y1, edges, zeros)
    y2, dexp = _sc_mid(p, y1, d, b1)
    q = _sc_scatter(y2, edges, zeros)

    q128 = q.reshape(NC, NP // 8, 8 * DH)
    y2_128 = y2.reshape(NP // 8, 8 * DH)
    dexp128 = dexp.reshape(NP // 8, 8 * DH)
    W2exp = (jnp.eye(8, dtype=jnp.float32)[:, None, :, None]
             * W2[None, :, None, :]).reshape(8 * DH, 8 * DF)
    b2exp = jnp.tile(b2, 8)

    out = _tc_out(q128, y2_128, dexp128, W2exp, b2exp)
    return out.reshape(N, DF)


# speedup vs baseline: 1.0704x; 1.0565x over previous
import functools

import jax
import jax.numpy as jnp
from jax import lax
from jax.experimental import pallas as pl
from jax.experimental.pallas import tpu as pltpu
from jax.experimental.pallas import tpu_sc as plsc

N = 10000
E = 320000
DF = 128
DH = 16
NP = 10240
NPP = 10112

NC = 2
NS = 16
NW = NC * NS
EPT = E // NW
B = 1000
NBF = EPT // B
TAIL = EPT - NBF * B
RPS = NP // NS
RPW = NP // NW
PIPE = 5

_mesh = plsc.VectorSubcoreMesh(core_axis_name="c", subcore_axis_name="s")
_sc_params = pltpu.CompilerParams(needs_layout_passes=False,
                                  use_tc_tiling_on_sc=False)


@functools.partial(
    pl.kernel,
    out_type=jax.ShapeDtypeStruct((NW * NPP,), jnp.float32),
    mesh=_mesh,
    compiler_params=_sc_params,
    scratch_types=[
        pltpu.VMEM((EPT,), jnp.int32),
        pltpu.VMEM((NPP,), jnp.float32),
    ],
)
def _sc_degree(edges_hbm, out_hbm, idx_v, deg_v):
    c = lax.axis_index("c")
    s = lax.axis_index("s")
    wid = c * NS + s
    pltpu.sync_copy(edges_hbm.at[1, pl.ds(wid * EPT, EPT)], idx_v)
    zeros = jnp.zeros((16,), jnp.float32)
    ones = jnp.ones((16,), jnp.float32)

    def zero_body(i, carry):
        deg_v[pl.ds(i * 16, 16)] = zeros
        return carry

    lax.fori_loop(0, NPP // 16, zero_body, 0, unroll=4)

    def edge_body(i, carry):
        dv = idx_v[pl.ds(i * 16, 16)]
        plsc.addupdate_scatter(deg_v, [dv], ones)
        return carry

    lax.fori_loop(0, EPT // 16, edge_body, 0, unroll=4)
    pltpu.sync_copy(deg_v, out_hbm.at[pl.ds(wid * NPP, NPP)])


@functools.partial(
    pl.kernel,
    out_type=jax.ShapeDtypeStruct((NC, NP, DH), jnp.float32),
    mesh=_mesh,
    compiler_params=_sc_params,
    scratch_types=[
        pltpu.VMEM((EPT,), jnp.int32),
        pltpu.VMEM((EPT,), jnp.int32),
        pltpu.VMEM((PIPE, B, DH), jnp.float32),
        pltpu.VMEM((max(TAIL, 16), DH), jnp.float32),
        pltpu.VMEM_SHARED((NP, DH), jnp.float32),
        pltpu.SemaphoreType.DMA((PIPE,)),
        pltpu.SemaphoreType.DMA((PIPE,)),
    ],
)
def _sc_scatter(y_hbm, edges_hbm, zero_hbm, out_hbm,
                src_v, dst_v, rows_v, tail_v, acc_sh, gsem, wsem):
    c = lax.axis_index("c")
    s = lax.axis_index("s")
    wid = c * NS + s

    cp_src = pltpu.async_copy(edges_hbm.at[0, pl.ds(wid * EPT, EPT)], src_v,
                              gsem.at[0])
    cp_dst = pltpu.async_copy(edges_hbm.at[1, pl.ds(wid * EPT, EPT)], dst_v,
                              wsem.at[0])
    cp_zero = pltpu.async_copy(zero_hbm.at[pl.ds(s * RPS, RPS)],
                               acc_sh.at[pl.ds(s * RPS, RPS)], wsem.at[1])

    def start_gather(j):
        slot = j % PIPE
        return pltpu.async_copy(y_hbm.at[src_v.at[pl.ds(j * B, B)]],
                                rows_v.at[slot], gsem.at[slot])

    def start_scatter(j):
        slot = j % PIPE
        return pltpu.async_copy(rows_v.at[slot],
                                acc_sh.at[dst_v.at[pl.ds(j * B, B)]],
                                wsem.at[slot], add=True)

    gathers = {}
    scatters = {}
    cp_src.wait()
    for j in range(min(PIPE, NBF)):
        gathers[j] = start_gather(j)
    cp_dst.wait()
    cp_zero.wait()
    plsc.subcore_barrier()
    for j in range(NBF):
        gathers.pop(j).wait()
        scatters[j] = start_scatter(j)
        nj = j + PIPE
        if nj < NBF:
            scatters.pop(nj - PIPE).wait()
            gathers[nj] = start_gather(nj)
    if TAIL:
        pltpu.async_copy(y_hbm.at[src_v.at[pl.ds(NBF * B, TAIL)]], tail_v,
                         gsem.at[0]).wait()
        pltpu.sync_copy(tail_v, acc_sh.at[dst_v.at[pl.ds(NBF * B, TAIL)]],
                        add=True)
    for j in list(scatters):
        scatters.pop(j).wait()

    plsc.subcore_barrier()
    pltpu.sync_copy(acc_sh.at[pl.ds(s * RPS, RPS)],
                    out_hbm.at[c, pl.ds(s * RPS, RPS)])


@functools.partial(
    pl.kernel,
    out_type=[jax.ShapeDtypeStruct((NP, DH), jnp.float32),
              jax.ShapeDtypeStruct((NP, DH), jnp.float32)],
    mesh=_mesh,
    compiler_params=_sc_params,
    scratch_types=[
        pltpu.VMEM((RPW, DH), jnp.float32),
        pltpu.VMEM((RPW, DH), jnp.float32),
        pltpu.VMEM((RPW, DH), jnp.float32),
        pltpu.VMEM((RPW,), jnp.float32),
        pltpu.VMEM((DH,), jnp.float32),
        pltpu.VMEM((RPW, DH), jnp.float32),
        pltpu.VMEM((RPW, DH), jnp.float32),
        pltpu.SemaphoreType.DMA((5,)),
    ],
)
def _sc_mid(p_hbm, y1_hbm, d_hbm, b1_hbm, out_hbm, dexp_hbm,
            p0_v, p1_v, y1_v, d_v, b1_v, y2_v, dexp_v, sems):
    c = lax.axis_index("c")
    s = lax.axis_index("s")
    wid = c * NS + s
    base = wid * RPW
    cps = [
        pltpu.async_copy(p_hbm.at[0, pl.ds(base, RPW)], p0_v, sems.at[0]),
        pltpu.async_copy(p_hbm.at[1, pl.ds(base, RPW)], p1_v, sems.at[1]),
        pltpu.async_copy(y1_hbm.at[pl.ds(base, RPW)], y1_v, sems.at[2]),
        pltpu.async_copy(d_hbm.at[pl.ds(base, RPW)], d_v, sems.at[3]),
        pltpu.async_copy(b1_hbm, b1_v, sems.at[4]),
    ]
    for cp in cps:
        cp.wait()
    b1 = b1_v[pl.ds(0, DH)]

    def row_body(r, carry):
        agg = p0_v[r] + p1_v[r] + y1_v[r]
        dval = plsc.load_gather(d_v, [jnp.full((16,), r, jnp.int32)])
        h = jnp.maximum(agg * dval + b1, 0.0)
        y2_v[r] = h * dval
        dexp_v[r] = dval
        return carry

    lax.fori_loop(0, RPW, row_body, 0, unroll=4)
    pltpu.sync_copy(y2_v, out_hbm.at[pl.ds(base, RPW)])
    pltpu.sync_copy(dexp_v, dexp_hbm.at[pl.ds(base, RPW)])


def _tc_matmul(x, W1):
    def body(x_ref, w_ref, out_ref):
        out_ref[...] = jnp.dot(x_ref[...], w_ref[...],
                               preferred_element_type=jnp.float32)

    return pl.pallas_call(
        body,
        out_shape=jax.ShapeDtypeStruct((N, DH), jnp.float32),
    )(x, W1)


def _tc_scale(degp, xw):
    def body(degp_ref, xw_ref, y_ref, d_ref):
        deg = jnp.full((N,), 1.0, jnp.float32)
        for w in range(NW):
            deg = deg + degp_ref[pl.ds(w * NPP, N)]
        d = lax.rsqrt(deg)
        y = xw_ref[...] * d[:, None]
        d_ref[...] = jnp.concatenate([d, jnp.ones((NP - N,), jnp.float32)])
        y_ref[...] = jnp.concatenate(
            [y, jnp.zeros((NP - N, DH), jnp.float32)], axis=0)

    return pl.pallas_call(
        body,
        out_shape=[jax.ShapeDtypeStruct((NP, DH), jnp.float32),
                   jax.ShapeDtypeStruct((NP,), jnp.float32)],
    )(degp, xw)


def _tc_out(q128, y2_128, dexp128, W2exp, b2exp):
    NR8 = N // 8

    def body(q_ref, y2_ref, de_ref, w_ref, b_ref, out_ref):
        t = (q_ref[0, :NR8] + q_ref[1, :NR8] + y2_ref[:NR8]) * de_ref[:NR8]
        for k in range(8):
            out_ref[:, k, :] = (
                jnp.dot(t, w_ref[:, k * DF:(k + 1) * DF],
                        preferred_element_type=jnp.float32)
                + b_ref[pl.ds(k * DF, DF)][None, :])

    return pl.pallas_call(
        body,
        out_shape=jax.ShapeDtypeStruct((NR8, 8, DF), jnp.float32),
    )(q128, y2_128, dexp128, W2exp, b2exp)


def kernel(x, edge_index, W1, b1, W2, b2):
    edges = edge_index.astype(jnp.int32)
    zeros = jnp.zeros((NP, DH), jnp.float32)

    xw = _tc_matmul(x, W1)
    degp = _sc_degree(edges)
    y1, d = _tc_scale(degp, xw)
    p = _sc_scatter(y1, edges, zeros)
    y2, dexp = _sc_mid(p, y1, d, b1)
    q = _sc_scatter(y2, edges, zeros)

    q128 = q.reshape(NC, NP // 8, 8 * DH)
    y2_128 = y2.reshape(NP // 8, 8 * DH)
    dexp128 = dexp.reshape(NP // 8, 8 * DH)
    W2exp = (jnp.eye(8, dtype=jnp.float32)[:, None, :, None]
             * W2[None, :, None, :]).reshape(8 * DH, 8 * DF)
    b2exp = jnp.tile(b2, 8)

    out = _tc_out(q128, y2_128, dexp128, W2exp, b2exp)
    return out.reshape(N, DF)


# speedup vs baseline: 1.0810x; 1.0098x over previous
import functools

import jax
import jax.numpy as jnp
from jax import lax
from jax.experimental import pallas as pl
from jax.experimental.pallas import tpu as pltpu
from jax.experimental.pallas import tpu_sc as plsc

N = 10000
E = 320000
DF = 128
DH = 16
NP = 10240
NPP = 10112

NC = 2
NS = 16
NW = NC * NS
EPT = E // NW
B = 1000
NBF = EPT // B
TAIL = EPT - NBF * B
RPS = NP // NS
RPW = NP // NW
PIPE = 5

_mesh = plsc.VectorSubcoreMesh(core_axis_name="c", subcore_axis_name="s")
_sc_params = pltpu.CompilerParams(needs_layout_passes=False,
                                  use_tc_tiling_on_sc=False)


@functools.partial(
    pl.kernel,
    out_type=jax.ShapeDtypeStruct((NW * NPP,), jnp.float32),
    mesh=_mesh,
    compiler_params=_sc_params,
    scratch_types=[
        pltpu.VMEM((EPT,), jnp.int32),
        pltpu.VMEM((NPP,), jnp.float32),
        pltpu.SemaphoreType.DMA((1,)),
    ],
)
def _sc_degree(edges_hbm, out_hbm, idx_v, deg_v, sem):
    c = lax.axis_index("c")
    s = lax.axis_index("s")
    wid = c * NS + s
    cp_idx = pltpu.async_copy(edges_hbm.at[1, pl.ds(wid * EPT, EPT)], idx_v,
                              sem.at[0])
    zeros = jnp.zeros((16,), jnp.float32)
    ones = jnp.ones((16,), jnp.float32)

    def zero_body(i, carry):
        deg_v[pl.ds(i * 16, 16)] = zeros
        return carry

    lax.fori_loop(0, NPP // 16, zero_body, 0, unroll=4)
    cp_idx.wait()

    def edge_body(i, carry):
        dv = idx_v[pl.ds(i * 16, 16)]
        plsc.addupdate_scatter(deg_v, [dv], ones)
        return carry

    lax.fori_loop(0, EPT // 16, edge_body, 0, unroll=4)
    pltpu.sync_copy(deg_v, out_hbm.at[pl.ds(wid * NPP, NPP)])


@functools.partial(
    pl.kernel,
    out_type=jax.ShapeDtypeStruct((NC, NP, DH), jnp.float32),
    mesh=_mesh,
    compiler_params=_sc_params,
    scratch_types=[
        pltpu.VMEM((EPT,), jnp.int32),
        pltpu.VMEM((EPT,), jnp.int32),
        pltpu.VMEM((PIPE, B, DH), jnp.float32),
        pltpu.VMEM((max(TAIL, 16), DH), jnp.float32),
        pltpu.VMEM_SHARED((NP, DH), jnp.float32),
        pltpu.SemaphoreType.DMA((PIPE,)),
        pltpu.SemaphoreType.DMA((PIPE,)),
    ],
)
def _sc_scatter(y_hbm, edges_hbm, zero_hbm, out_hbm,
                src_v, dst_v, rows_v, tail_v, acc_sh, gsem, wsem):
    c = lax.axis_index("c")
    s = lax.axis_index("s")
    wid = c * NS + s

    cp_src = pltpu.async_copy(edges_hbm.at[0, pl.ds(wid * EPT, EPT)], src_v,
                              gsem.at[0])
    cp_dst = pltpu.async_copy(edges_hbm.at[1, pl.ds(wid * EPT, EPT)], dst_v,
                              wsem.at[0])
    cp_zero = pltpu.async_copy(zero_hbm.at[pl.ds(s * RPS, RPS)],
                               acc_sh.at[pl.ds(s * RPS, RPS)], wsem.at[1])

    def start_gather(j):
        slot = j % PIPE
        return pltpu.async_copy(y_hbm.at[src_v.at[pl.ds(j * B, B)]],
                                rows_v.at[slot], gsem.at[slot])

    def start_scatter(j):
        slot = j % PIPE
        return pltpu.async_copy(rows_v.at[slot],
                                acc_sh.at[dst_v.at[pl.ds(j * B, B)]],
                                wsem.at[slot], add=True)

    gathers = {}
    scatters = {}
    cp_src.wait()
    for j in range(min(PIPE, NBF)):
        gathers[j] = start_gather(j)
    cp_dst.wait()
    cp_zero.wait()
    plsc.subcore_barrier()
    for j in range(NBF):
        gathers.pop(j).wait()
        scatters[j] = start_scatter(j)
        nj = j + PIPE
        if nj < NBF:
            scatters.pop(nj - PIPE).wait()
            gathers[nj] = start_gather(nj)
    if TAIL:
        pltpu.async_copy(y_hbm.at[src_v.at[pl.ds(NBF * B, TAIL)]], tail_v,
                         gsem.at[0]).wait()
        pltpu.sync_copy(tail_v, acc_sh.at[dst_v.at[pl.ds(NBF * B, TAIL)]],
                        add=True)
    for j in list(scatters):
        scatters.pop(j).wait()

    plsc.subcore_barrier()
    pltpu.sync_copy(acc_sh.at[pl.ds(s * RPS, RPS)],
                    out_hbm.at[c, pl.ds(s * RPS, RPS)])


@functools.partial(
    pl.kernel,
    out_type=[jax.ShapeDtypeStruct((NP, DH), jnp.float32),
              jax.ShapeDtypeStruct((NP, DH), jnp.float32)],
    mesh=_mesh,
    compiler_params=_sc_params,
    scratch_types=[
        pltpu.VMEM((RPW, DH), jnp.float32),
        pltpu.VMEM((RPW, DH), jnp.float32),
        pltpu.VMEM((RPW, DH), jnp.float32),
        pltpu.VMEM((RPW,), jnp.float32),
        pltpu.VMEM((DH,), jnp.float32),
        pltpu.VMEM((RPW, DH), jnp.float32),
        pltpu.VMEM((RPW, DH), jnp.float32),
        pltpu.SemaphoreType.DMA((5,)),
    ],
)
def _sc_mid(p_hbm, y1_hbm, d_hbm, b1_hbm, out_hbm, dexp_hbm,
            p0_v, p1_v, y1_v, d_v, b1_v, y2_v, dexp_v, sems):
    c = lax.axis_index("c")
    s = lax.axis_index("s")
    wid = c * NS + s
    base = wid * RPW
    cps = [
        pltpu.async_copy(p_hbm.at[0, pl.ds(base, RPW)], p0_v, sems.at[0]),
        pltpu.async_copy(p_hbm.at[1, pl.ds(base, RPW)], p1_v, sems.at[1]),
        pltpu.async_copy(y1_hbm.at[pl.ds(base, RPW)], y1_v, sems.at[2]),
        pltpu.async_copy(d_hbm.at[pl.ds(base, RPW)], d_v, sems.at[3]),
        pltpu.async_copy(b1_hbm, b1_v, sems.at[4]),
    ]
    for cp in cps:
        cp.wait()
    b1 = b1_v[pl.ds(0, DH)]

    def row_body(r, carry):
        agg = p0_v[r] + p1_v[r] + y1_v[r]
        dval = plsc.load_gather(d_v, [jnp.full((16,), r, jnp.int32)])
        h = jnp.maximum(agg * dval + b1, 0.0)
        y2_v[r] = h * dval
        dexp_v[r] = dval
        return carry

    lax.fori_loop(0, RPW, row_body, 0, unroll=4)
    cp_y2 = pltpu.async_copy(y2_v, out_hbm.at[pl.ds(base, RPW)], sems.at[0])
    cp_de = pltpu.async_copy(dexp_v, dexp_hbm.at[pl.ds(base, RPW)],
                             sems.at[1])
    cp_y2.wait()
    cp_de.wait()


def _tc_matmul(x, W1):
    def body(x_ref, w_ref, out_ref):
        out_ref[...] = jnp.dot(x_ref[...], w_ref[...],
                               preferred_element_type=jnp.float32)

    return pl.pallas_call(
        body,
        out_shape=jax.ShapeDtypeStruct((N, DH), jnp.float32),
    )(x, W1)


def _tc_scale(degp, xw):
    def body(degp_ref, xw_ref, y_ref, d_ref):
        deg = jnp.full((N,), 1.0, jnp.float32)
        for w in range(NW):
            deg = deg + degp_ref[pl.ds(w * NPP, N)]
        d = lax.rsqrt(deg)
        y = xw_ref[...] * d[:, None]
        d_ref[...] = jnp.concatenate([d, jnp.ones((NP - N,), jnp.float32)])
        y_ref[...] = jnp.concatenate(
            [y, jnp.zeros((NP - N, DH), jnp.float32)], axis=0)

    return pl.pallas_call(
        body,
        out_shape=[jax.ShapeDtypeStruct((NP, DH), jnp.float32),
                   jax.ShapeDtypeStruct((NP,), jnp.float32)],
    )(degp, xw)


def _tc_out(q128, y2_128, dexp128, W2exp, b2exp):
    NR8 = N // 8

    def body(q_ref, y2_ref, de_ref, w_ref, b_ref, out_ref):
        t = (q_ref[0, :NR8] + q_ref[1, :NR8] + y2_ref[:NR8]) * de_ref[:NR8]
        for k in range(8):
            out_ref[:, k, :] = (
                jnp.dot(t, w_ref[:, k * DF:(k + 1) * DF],
                        preferred_element_type=jnp.float32)
                + b_ref[pl.ds(k * DF, DF)][None, :])

    return pl.pallas_call(
        body,
        out_shape=jax.ShapeDtypeStruct((NR8, 8, DF), jnp.float32),
    )(q128, y2_128, dexp128, W2exp, b2exp)


def kernel(x, edge_index, W1, b1, W2, b2):
    edges = edge_index.astype(jnp.int32)
    zeros = jnp.zeros((NP, DH), jnp.float32)

    xw = _tc_matmul(x, W1)
    degp = _sc_degree(edges)
    y1, d = _tc_scale(degp, xw)
    p = _sc_scatter(y1, edges, zeros)
    y2, dexp = _sc_mid(p, y1, d, b1)
    q = _sc_scatter(y2, edges, zeros)

    q128 = q.reshape(NC, NP // 8, 8 * DH)
    y2_128 = y2.reshape(NP // 8, 8 * DH)
    dexp128 = dexp.reshape(NP // 8, 8 * DH)
    W2exp = (jnp.eye(8, dtype=jnp.float32)[:, None, :, None]
             * W2[None, :, None, :]).reshape(8 * DH, 8 * DF)
    b2exp = jnp.tile(b2, 8)

    out = _tc_out(q128, y2_128, dexp128, W2exp, b2exp)
    return out.reshape(N, DF)
